# Initial kernel scaffold; baseline (speedup 1.0000x reference)
import jax
import jax.numpy as jnp
from jax.experimental import pallas as pl


def kernel(on_surface_sample, extrinsics, latent_codes, probabilities, sphere_points, params):
    raise NotImplementedError("write your pallas kernel here")


# speedup vs baseline: 1.0815x; 1.0815x over previous
import functools

import jax
import jax.numpy as jnp
from jax.experimental import pallas as pl
from jax.experimental.pallas import tpu as pltpu

B = 2
N_PTS = 100000
P = 8
LAT = 128
HID = 256
NLAYERS = 6
W0 = 30.0
K_SEL = 1024
M_ROWS = 2 * K_SEL
CH = 2048
NCH = 49
N_PAD = CH * NCH
NEG_BIG = -3.4e38


def _quat_params(ext):
    constants = ext[..., 0]
    scales = jax.nn.softplus(ext[..., 1:4]) + 0.1
    q = ext[..., 4:8]
    q = q / (jnp.linalg.norm(q, axis=-1, keepdims=True) + 1e-8)
    w = q[..., 0]; x = q[..., 1]; y = q[..., 2]; z = q[..., 3]
    R = jnp.stack([
        1.0 - 2.0 * (y * y + z * z), 2.0 * (x * y - w * z), 2.0 * (x * z + w * y),
        2.0 * (x * y + w * z), 1.0 - 2.0 * (x * x + z * z), 2.0 * (y * z - w * x),
        2.0 * (x * z - w * y), 2.0 * (y * z + w * x), 1.0 - 2.0 * (x * x + y * y)
    ], axis=-1)
    Rb = R.astype(jnp.bfloat16).astype(jnp.float32)
    ptab = jnp.concatenate([Rb, scales, ext[..., 8:11], constants[..., None]],
                           axis=-1).reshape(B * P, 16)
    return ptab, constants, scales, R.reshape(B, P, 3, 3), ext[..., 8:11]


def _b16(x):
    return x.astype(jnp.bfloat16).astype(jnp.float32)


def _score_body(ptab_ref, xT_ref, probT_ref, out_ref):
    b = pl.program_id(0)
    c = pl.program_id(1)
    col = c * CH + jax.lax.broadcasted_iota(jnp.int32, (1, CH), 1)
    in_range = col < N_PTS
    xs = [xT_ref[:, i, :] for i in range(3)]
    rows = []
    for p in range(P):
        row = b * P + p
        R = [ptab_ref[row, j] for j in range(9)]
        s = [ptab_ref[row, 9 + j] for j in range(3)]
        cc = [ptab_ref[row, 12 + j] for j in range(3)]
        d = [_b16(xs[i] - cc[i]) for i in range(3)]
        dist2 = None
        for j in range(3):
            lp = (R[0 * 3 + j] * d[0] + R[1 * 3 + j] * d[1] + R[2 * 3 + j] * d[2]) / s[j]
            lp2 = lp * lp
            dist2 = lp2 if dist2 is None else dist2 + lp2
        dist = jnp.sqrt(dist2)
        valid = dist < 1.0
        sc = jnp.where(valid, probT_ref[:, p, :], -dist)
        rows.append(jnp.where(in_range, sc, NEG_BIG))
    out_ref[0] = jnp.concatenate(rows, axis=0)


def _decode_body(ptab_ref, bout_ref, xgT_ref, sphT_ref, latT_ref,
                 w0t_ref, wt_ref, bcol_ref, wmt_ref, wcol_ref,
                 cs_ref, vis_ref, sdf_ref, scal_ref, pw_ref, wsdf_ref):
    b = pl.program_id(0)
    lat = latT_ref[0]
    mods = []
    for i in range(NLAYERS):
        mods.append(jax.nn.relu(jnp.dot(
            wmt_ref[i], lat, preferred_element_type=jnp.float32)))
    w_rows = []
    s_rows = []
    for p in range(P):
        row = b * P + p
        R = [ptab_ref[row, j] for j in range(9)]
        s = [ptab_ref[row, 9 + j] for j in range(3)]
        cc = [ptab_ref[row, 12 + j] for j in range(3)]
        const = ptab_ref[row, 15]
        d = [_b16(xgT_ref[0, p, i:i + 1, :] - cc[i]) for i in range(3)]
        nr = [_b16(xgT_ref[0, p, 3 + i:4 + i, :]) for i in range(3)]
        sp = [sphT_ref[0, j:j + 1, :] for j in range(3)]
        lp = [(R[0 * 3 + j] * d[0] + R[1 * 3 + j] * d[1] + R[2 * 3 + j] * d[2]) / s[j]
              for j in range(3)]
        ln = [R[0 * 3 + j] * nr[0] + R[1 * 3 + j] * nr[1] + R[2 * 3 + j] * nr[2]
              for j in range(3)]
        cs = ([jnp.concatenate([lp[j], sp[j]], axis=1) for j in range(3)] +
              [jnp.concatenate([ln[j], sp[j]], axis=1) for j in range(3)])
        csl = [_b16(cs[j] * s[j]) for j in range(3)]
        csn = [_b16(cs[3 + j]) for j in range(3)]
        gp = [R[i * 3 + 0] * csl[0] + R[i * 3 + 1] * csl[1] + R[i * 3 + 2] * csl[2] + cc[i]
              for i in range(3)]
        gn = [R[i * 3 + 0] * csn[0] + R[i * 3 + 1] * csn[1] + R[i * 3 + 2] * csn[2]
              for i in range(3)]
        csb = [_b16(cs[j]) for j in range(3)]
        pre = (w0t_ref[:, 0:1] * csb[0] + w0t_ref[:, 1:2] * csb[1]
               + w0t_ref[:, 2:3] * csb[2]) + bcol_ref[0]
        hT = jnp.sin(W0 * pre) * mods[0][:, p:p + 1]
        for i in range(1, NLAYERS):
            pre = jnp.dot(wt_ref[i - 1], hT.astype(jnp.bfloat16),
                          preferred_element_type=jnp.float32) + bcol_ref[i]
            hT = jnp.sin(W0 * pre) * mods[i][:, p:p + 1]
        sdf = jnp.sum(_b16(hT) * wcol_ref[...], axis=0, keepdims=True) + bout_ref[0, 0]
        dv = [_b16(gp[i] - cc[i]) for i in range(3)]
        scaled = None
        for j in range(3):
            e = (R[0 * 3 + j] * dv[0] + R[1 * 3 + j] * dv[1] + R[2 * 3 + j] * dv[2]) / s[j]
            e2 = e * e
            scaled = e2 if scaled is None else scaled + e2
        wgt = jnp.abs(const) * jnp.exp(-0.5 * scaled)
        for j in range(3):
            cs_ref[0, j, p] = cs[j].reshape(M_ROWS)
            cs_ref[0, 3 + j, p] = cs[3 + j].reshape(M_ROWS)
            vis_ref[0, j, p] = gp[j].reshape(M_ROWS)
            vis_ref[0, 3 + j, p] = gn[j].reshape(M_ROWS)
        sdf_ref[0, p] = sdf.reshape(M_ROWS)
        scal_ref[0, p] = scaled.reshape(M_ROWS)
        w_rows.append(wgt)
        s_rows.append(sdf)
    pwn = w_rows[0]
    for p in range(1, P):
        pwn = pwn + w_rows[p]
    mask = pwn == 0.0
    denom = jnp.where(mask, 1.0, pwn)
    wsdf = None
    for p in range(P):
        wn = jnp.where(mask, 0.0, w_rows[p] / denom)
        pw_ref[0, p] = wn.reshape(M_ROWS)
        t = wn * s_rows[p]
        wsdf = t if wsdf is None else wsdf + t
    wsdf = jnp.where(mask, 1.0, wsdf)
    wsdf_ref[0] = wsdf


def kernel(on_surface_sample, extrinsics, latent_codes, probabilities, sphere_points, params):
    ptab, constants, scales, rotations, centers = _quat_params(extrinsics)

    xT = jnp.transpose(on_surface_sample, (0, 2, 1))
    xT_pad = jnp.pad(xT, ((0, 0), (0, 0), (0, N_PAD - N_PTS)))
    probT = jnp.transpose(probabilities, (0, 2, 1))
    probT_pad = jnp.pad(probT, ((0, 0), (0, 0), (0, N_PAD - N_PTS)))

    scores = pl.pallas_call(
        _score_body,
        grid=(B, NCH),
        in_specs=[
            pl.BlockSpec(memory_space=pltpu.SMEM),
            pl.BlockSpec((1, 6, CH), lambda b, c: (b, 0, c)),
            pl.BlockSpec((1, P, CH), lambda b, c: (b, 0, c)),
        ],
        out_specs=pl.BlockSpec((1, P, CH), lambda b, c: (b, 0, c)),
        out_shape=jax.ShapeDtypeStruct((B, P, N_PAD), jnp.float32),
    )(ptab, xT_pad, probT_pad)

    idx = jax.lax.top_k(scores, K_SEL)[1]
    xgT = jnp.take_along_axis(
        xT[:, None], idx[:, :, None, :].astype(jnp.int32), axis=3)

    sphT = jnp.transpose(sphere_points, (0, 2, 1))
    latT = jnp.transpose(latent_codes, (0, 2, 1)).astype(jnp.bfloat16)
    w0t = jnp.transpose(params['W'][0], (1, 0)).astype(jnp.bfloat16).astype(jnp.float32)
    wt = jnp.stack([jnp.transpose(params['W'][i], (1, 0))
                    for i in range(1, NLAYERS)]).astype(jnp.bfloat16)
    bcol = jnp.stack(params['b'])[:, :, None]
    wmt = jnp.stack([jnp.transpose(params['Wm'][i], (1, 0))
                     for i in range(NLAYERS)]).astype(jnp.bfloat16)
    wcol = params['Wout'].astype(jnp.bfloat16).astype(jnp.float32)
    bout2 = params['bout'].reshape(1, 1)

    const_spec = lambda shape: pl.BlockSpec(shape, lambda b: (0,) * len(shape))
    outs = pl.pallas_call(
        _decode_body,
        grid=(B,),
        in_specs=[
            pl.BlockSpec(memory_space=pltpu.SMEM),
            pl.BlockSpec(memory_space=pltpu.SMEM),
            pl.BlockSpec((1, P, 6, K_SEL), lambda b: (b, 0, 0, 0)),
            pl.BlockSpec((1, 3, K_SEL), lambda b: (b, 0, 0)),
            pl.BlockSpec((1, LAT, P), lambda b: (b, 0, 0)),
            const_spec((HID, 3)),
            const_spec((NLAYERS - 1, HID, HID)),
            const_spec((NLAYERS, HID, 1)),
            const_spec((NLAYERS, HID, LAT)),
            const_spec((HID, 1)),
        ],
        out_specs=[
            pl.BlockSpec((1, 6, P, M_ROWS), lambda b: (b, 0, 0, 0)),
            pl.BlockSpec((1, 6, P, M_ROWS), lambda b: (b, 0, 0, 0)),
            pl.BlockSpec((1, P, M_ROWS), lambda b: (b, 0, 0)),
            pl.BlockSpec((1, P, M_ROWS), lambda b: (b, 0, 0)),
            pl.BlockSpec((1, P, M_ROWS), lambda b: (b, 0, 0)),
            pl.BlockSpec((1, 1, M_ROWS), lambda b: (b, 0, 0)),
        ],
        out_shape=[
            jax.ShapeDtypeStruct((B, 6, P, M_ROWS), jnp.float32),
            jax.ShapeDtypeStruct((B, 6, P, M_ROWS), jnp.float32),
            jax.ShapeDtypeStruct((B, P, M_ROWS), jnp.float32),
            jax.ShapeDtypeStruct((B, P, M_ROWS), jnp.float32),
            jax.ShapeDtypeStruct((B, P, M_ROWS), jnp.float32),
            jax.ShapeDtypeStruct((B, 1, M_ROWS), jnp.float32),
        ],
    )(ptab, bout2, xgT, sphT, latT, w0t, wt, bcol, wmt, wcol)
    csT_o, visT_o, sdf_o, scal_o, pw_o, wsdf_o = outs

    coords_select = jnp.transpose(csT_o, (0, 3, 2, 1))
    coords_select_vis = jnp.transpose(visT_o, (0, 3, 2, 1))
    coords_input = coords_select[..., :3]
    patch_sdfs = jnp.transpose(sdf_o, (0, 2, 1))
    scaled_distance = jnp.transpose(scal_o, (0, 2, 1))
    patch_weight = jnp.transpose(pw_o, (0, 2, 1))
    weighted_sdf = jnp.transpose(wsdf_o, (0, 2, 1))
    ext_out = scales[:, :, 0]
    return (weighted_sdf, coords_select_vis, coords_select, coords_input,
            patch_weight, patch_sdfs, scaled_distance, ext_out, centers)


# speedup vs baseline: 1.1100x; 1.0263x over previous
import functools

import jax
import jax.numpy as jnp
from jax.experimental import pallas as pl
from jax.experimental.pallas import tpu as pltpu

B = 2
N_PTS = 100000
P = 8
LAT = 128
HID = 256
NLAYERS = 6
W0 = 30.0
K_SEL = 1024
M_ROWS = 2 * K_SEL
CH = 2048
NCH = 49
N_PAD = CH * NCH
NEG_BIG = -3.4e38


def _quat_params(ext):
    constants = ext[..., 0]
    scales = jax.nn.softplus(ext[..., 1:4]) + 0.1
    q = ext[..., 4:8]
    q = q / (jnp.linalg.norm(q, axis=-1, keepdims=True) + 1e-8)
    w = q[..., 0]; x = q[..., 1]; y = q[..., 2]; z = q[..., 3]
    R = jnp.stack([
        1.0 - 2.0 * (y * y + z * z), 2.0 * (x * y - w * z), 2.0 * (x * z + w * y),
        2.0 * (x * y + w * z), 1.0 - 2.0 * (x * x + z * z), 2.0 * (y * z - w * x),
        2.0 * (x * z - w * y), 2.0 * (y * z + w * x), 1.0 - 2.0 * (x * x + y * y)
    ], axis=-1)
    Rb = R.astype(jnp.bfloat16).astype(jnp.float32)
    ptab = jnp.concatenate([Rb, scales, ext[..., 8:11], constants[..., None]],
                           axis=-1).reshape(B * P, 16)
    return ptab, constants, scales, R.reshape(B, P, 3, 3), ext[..., 8:11]


def _b16(x):
    return x.astype(jnp.bfloat16).astype(jnp.float32)


def _score_body(ptab_ref, xT_ref, probT_ref, out_ref):
    b = pl.program_id(0)
    c = pl.program_id(1)
    col = c * CH + jax.lax.broadcasted_iota(jnp.int32, (1, CH), 1)
    in_range = col < N_PTS
    xs = [xT_ref[:, i, :] for i in range(3)]
    rows = []
    for p in range(P):
        row = b * P + p
        R = [ptab_ref[row, j] for j in range(9)]
        s = [ptab_ref[row, 9 + j] for j in range(3)]
        cc = [ptab_ref[row, 12 + j] for j in range(3)]
        d = [_b16(xs[i] - cc[i]) for i in range(3)]
        dist2 = None
        for j in range(3):
            lp = (R[0 * 3 + j] * d[0] + R[1 * 3 + j] * d[1] + R[2 * 3 + j] * d[2]) / s[j]
            lp2 = lp * lp
            dist2 = lp2 if dist2 is None else dist2 + lp2
        dist = jnp.sqrt(dist2)
        valid = dist < 1.0
        sc = jnp.where(valid, probT_ref[:, p, :], -dist)
        rows.append(jnp.where(in_range, sc, NEG_BIG))
    out_ref[0] = jnp.concatenate(rows, axis=0)


def _decode_body(ptab_ref, bout_ref, xgT_ref, sphT_ref, latT_ref,
                 w0t_ref, wt_ref, bcol_ref, wmt_ref, wcol_ref,
                 cs_ref, vis_ref, sdf_ref, scal_ref, pw_ref, wsdf_ref):
    b = pl.program_id(0)
    lat = latT_ref[0]
    mods = []
    for i in range(NLAYERS):
        mods.append(jax.nn.relu(jnp.dot(
            wmt_ref[i], lat, preferred_element_type=jnp.float32)))
    w_rows = []
    s_rows = []
    for p in range(P):
        row = b * P + p
        R = [ptab_ref[row, j] for j in range(9)]
        s = [ptab_ref[row, 9 + j] for j in range(3)]
        cc = [ptab_ref[row, 12 + j] for j in range(3)]
        const = ptab_ref[row, 15]
        d = [_b16(xgT_ref[0, p, i:i + 1, :] - cc[i]) for i in range(3)]
        nr = [_b16(xgT_ref[0, p, 3 + i:4 + i, :]) for i in range(3)]
        sp = [sphT_ref[0, j:j + 1, :] for j in range(3)]
        lp = [(R[0 * 3 + j] * d[0] + R[1 * 3 + j] * d[1] + R[2 * 3 + j] * d[2]) / s[j]
              for j in range(3)]
        ln = [R[0 * 3 + j] * nr[0] + R[1 * 3 + j] * nr[1] + R[2 * 3 + j] * nr[2]
              for j in range(3)]
        cs = ([jnp.concatenate([lp[j], sp[j]], axis=1) for j in range(3)] +
              [jnp.concatenate([ln[j], sp[j]], axis=1) for j in range(3)])
        csl = [_b16(cs[j] * s[j]) for j in range(3)]
        csn = [_b16(cs[3 + j]) for j in range(3)]
        gp = [R[i * 3 + 0] * csl[0] + R[i * 3 + 1] * csl[1] + R[i * 3 + 2] * csl[2] + cc[i]
              for i in range(3)]
        gn = [R[i * 3 + 0] * csn[0] + R[i * 3 + 1] * csn[1] + R[i * 3 + 2] * csn[2]
              for i in range(3)]
        csb = [_b16(cs[j]) for j in range(3)]
        pre = (w0t_ref[:, 0:1] * csb[0] + w0t_ref[:, 1:2] * csb[1]
               + w0t_ref[:, 2:3] * csb[2]) + bcol_ref[0]
        hT = jnp.sin(W0 * pre) * mods[0][:, p:p + 1]
        for i in range(1, NLAYERS):
            pre = jnp.dot(wt_ref[i - 1], hT.astype(jnp.bfloat16),
                          preferred_element_type=jnp.float32) + bcol_ref[i]
            hT = jnp.sin(W0 * pre) * mods[i][:, p:p + 1]
        sdf = jnp.sum(_b16(hT) * wcol_ref[...], axis=0, keepdims=True) + bout_ref[0, 0]
        dv = [_b16(gp[i] - cc[i]) for i in range(3)]
        scaled = None
        for j in range(3):
            e = (R[0 * 3 + j] * dv[0] + R[1 * 3 + j] * dv[1] + R[2 * 3 + j] * dv[2]) / s[j]
            e2 = e * e
            scaled = e2 if scaled is None else scaled + e2
        wgt = jnp.abs(const) * jnp.exp(-0.5 * scaled)
        for j in range(3):
            cs_ref[0, j, p] = cs[j].reshape(M_ROWS)
            cs_ref[0, 3 + j, p] = cs[3 + j].reshape(M_ROWS)
            vis_ref[0, j, p] = gp[j].reshape(M_ROWS)
            vis_ref[0, 3 + j, p] = gn[j].reshape(M_ROWS)
        sdf_ref[0, p] = sdf.reshape(M_ROWS)
        scal_ref[0, p] = scaled.reshape(M_ROWS)
        w_rows.append(wgt)
        s_rows.append(sdf)
    pwn = w_rows[0]
    for p in range(1, P):
        pwn = pwn + w_rows[p]
    mask = pwn == 0.0
    denom = jnp.where(mask, 1.0, pwn)
    wsdf = None
    for p in range(P):
        wn = jnp.where(mask, 0.0, w_rows[p] / denom)
        pw_ref[0, p] = wn.reshape(M_ROWS)
        t = wn * s_rows[p]
        wsdf = t if wsdf is None else wsdf + t
    wsdf = jnp.where(mask, 1.0, wsdf)
    wsdf_ref[0] = wsdf


def kernel(on_surface_sample, extrinsics, latent_codes, probabilities, sphere_points, params):
    ptab, constants, scales, rotations, centers = _quat_params(extrinsics)

    xT = jnp.transpose(on_surface_sample, (0, 2, 1))
    xT_pad = jnp.pad(xT, ((0, 0), (0, 0), (0, N_PAD - N_PTS)))
    probT = jnp.transpose(probabilities, (0, 2, 1))
    probT_pad = jnp.pad(probT, ((0, 0), (0, 0), (0, N_PAD - N_PTS)))

    scores = pl.pallas_call(
        _score_body,
        grid=(B, NCH),
        in_specs=[
            pl.BlockSpec(memory_space=pltpu.SMEM),
            pl.BlockSpec((1, 6, CH), lambda b, c: (b, 0, c)),
            pl.BlockSpec((1, P, CH), lambda b, c: (b, 0, c)),
        ],
        out_specs=pl.BlockSpec((1, P, CH), lambda b, c: (b, 0, c)),
        out_shape=jax.ShapeDtypeStruct((B, P, N_PAD), jnp.float32),
    )(ptab, xT_pad, probT_pad)

    NSPLIT = 4
    CW = N_PAD // NSPLIT
    vc, ic = jax.lax.top_k(scores.reshape(B, P, NSPLIT, CW), K_SEL)
    ic = ic + (jnp.arange(NSPLIT, dtype=jnp.int32) * CW)[None, None, :, None]
    vc = vc.reshape(B, P, NSPLIT * K_SEL)
    ic = ic.reshape(B, P, NSPLIT * K_SEL)
    i2 = jax.lax.top_k(vc, K_SEL)[1]
    idx = jnp.take_along_axis(ic, i2, axis=2)
    xgT = jnp.take_along_axis(
        xT[:, None], idx[:, :, None, :].astype(jnp.int32), axis=3)

    sphT = jnp.transpose(sphere_points, (0, 2, 1))
    latT = jnp.transpose(latent_codes, (0, 2, 1)).astype(jnp.bfloat16)
    w0t = jnp.transpose(params['W'][0], (1, 0)).astype(jnp.bfloat16).astype(jnp.float32)
    wt = jnp.stack([jnp.transpose(params['W'][i], (1, 0))
                    for i in range(1, NLAYERS)]).astype(jnp.bfloat16)
    bcol = jnp.stack(params['b'])[:, :, None]
    wmt = jnp.stack([jnp.transpose(params['Wm'][i], (1, 0))
                     for i in range(NLAYERS)]).astype(jnp.bfloat16)
    wcol = params['Wout'].astype(jnp.bfloat16).astype(jnp.float32)
    bout2 = params['bout'].reshape(1, 1)

    const_spec = lambda shape: pl.BlockSpec(shape, lambda b: (0,) * len(shape))
    outs = pl.pallas_call(
        _decode_body,
        grid=(B,),
        in_specs=[
            pl.BlockSpec(memory_space=pltpu.SMEM),
            pl.BlockSpec(memory_space=pltpu.SMEM),
            pl.BlockSpec((1, P, 6, K_SEL), lambda b: (b, 0, 0, 0)),
            pl.BlockSpec((1, 3, K_SEL), lambda b: (b, 0, 0)),
            pl.BlockSpec((1, LAT, P), lambda b: (b, 0, 0)),
            const_spec((HID, 3)),
            const_spec((NLAYERS - 1, HID, HID)),
            const_spec((NLAYERS, HID, 1)),
            const_spec((NLAYERS, HID, LAT)),
            const_spec((HID, 1)),
        ],
        out_specs=[
            pl.BlockSpec((1, 6, P, M_ROWS), lambda b: (b, 0, 0, 0)),
            pl.BlockSpec((1, 6, P, M_ROWS), lambda b: (b, 0, 0, 0)),
            pl.BlockSpec((1, P, M_ROWS), lambda b: (b, 0, 0)),
            pl.BlockSpec((1, P, M_ROWS), lambda b: (b, 0, 0)),
            pl.BlockSpec((1, P, M_ROWS), lambda b: (b, 0, 0)),
            pl.BlockSpec((1, 1, M_ROWS), lambda b: (b, 0, 0)),
        ],
        out_shape=[
            jax.ShapeDtypeStruct((B, 6, P, M_ROWS), jnp.float32),
            jax.ShapeDtypeStruct((B, 6, P, M_ROWS), jnp.float32),
            jax.ShapeDtypeStruct((B, P, M_ROWS), jnp.float32),
            jax.ShapeDtypeStruct((B, P, M_ROWS), jnp.float32),
            jax.ShapeDtypeStruct((B, P, M_ROWS), jnp.float32),
            jax.ShapeDtypeStruct((B, 1, M_ROWS), jnp.float32),
        ],
    )(ptab, bout2, xgT, sphT, latT, w0t, wt, bcol, wmt, wcol)
    csT_o, visT_o, sdf_o, scal_o, pw_o, wsdf_o = outs

    coords_select = jnp.transpose(csT_o, (0, 3, 2, 1))
    coords_select_vis = jnp.transpose(visT_o, (0, 3, 2, 1))
    coords_input = coords_select[..., :3]
    patch_sdfs = jnp.transpose(sdf_o, (0, 2, 1))
    scaled_distance = jnp.transpose(scal_o, (0, 2, 1))
    patch_weight = jnp.transpose(pw_o, (0, 2, 1))
    weighted_sdf = jnp.transpose(wsdf_o, (0, 2, 1))
    ext_out = scales[:, :, 0]
    return (weighted_sdf, coords_select_vis, coords_select, coords_input,
            patch_weight, patch_sdfs, scaled_distance, ext_out, centers)
